# baseline (speedup 1.0000x reference)
import functools

import jax
import jax.numpy as jnp
from jax import lax
from jax.experimental import pallas as pl
from jax.experimental.pallas import tpu as pltpu
from jax.experimental.pallas import tpu_sc as plsc

INPUT_DIM = 768
EMBED_DIM = 256
HIDDEN = 512
NUM_EMB = 8192
BATCH = 4096
COMMIT = 0.25

BM = 512
NB = BATCH // BM
CHUNK = 1024
NCH = NUM_EMB // CHUNK

_BF = jnp.bfloat16


def _leaky(v):
    return jnp.where(v > 0, v, 0.01 * v)


def _dot(a, b):
    return jnp.dot(a.astype(_BF), b.astype(_BF),
                   preferred_element_type=jnp.float32)


def _enc_body(x_ref, we1_ref, be1_ref, we2_ref, be2_ref, cb_ref, cn_ref,
              idx_ref, ls_ref):
    h = _leaky(_dot(x_ref[...], we1_ref[...]) + be1_ref[...])
    z = _dot(h, we2_ref[...]) + be2_ref[...]
    znorm = jnp.sum(z * z, axis=1, keepdims=True)
    zm2b = (-2.0 * z).astype(_BF)
    runv = runc = None
    for c in range(NCH):
        sl = slice(c * CHUNK, (c + 1) * CHUNK)
        mm2 = lax.dot_general(zm2b, cb_ref[sl, :].astype(_BF),
                              (((1,), (1,)), ((), ())),
                              preferred_element_type=jnp.float32)
        s = (znorm + cn_ref[:, sl]) + mm2
        if c == 0:
            runv = s
            runc = jnp.zeros(s.shape, jnp.int32)
        else:
            upd = s < runv
            runv = jnp.where(upd, s, runv)
            runc = jnp.where(upd, jnp.int32(c), runc)
    gidx = runc * CHUNK + lax.broadcasted_iota(jnp.int32, runv.shape, 1)
    vmin = jnp.min(runv, axis=1, keepdims=True)
    cand = jnp.where(runv == vmin, gidx, jnp.int32(0x7FFFFFFF))
    idx_ref[...] = jnp.min(cand, axis=1)[:, None]
    ls_ref[...] = jnp.sum(vmin).reshape(1, 1, 1)


def _dec_body(zq_ref, wd1_ref, bd1_ref, wd2_ref, bd2_ref, xr_ref):
    h = _leaky(_dot(zq_ref[...], wd1_ref[...]) + bd1_ref[...])
    xr_ref[...] = _dot(h, wd2_ref[...]) + bd2_ref[...]


HALF = BATCH // 2
NBH = HALF // BM


def _sc_gather(codebook, idx):
    info = plsc.get_sparse_core_info()
    nw = info.num_cores * info.num_subcores
    b_per_w = HALF // nw
    mesh = plsc.VectorSubcoreMesh(core_axis_name="c", subcore_axis_name="s")

    @functools.partial(
        pl.kernel,
        out_type=jax.ShapeDtypeStruct((HALF, EMBED_DIM), jnp.float32),
        mesh=mesh,
        scratch_types=[
            pltpu.VMEM((b_per_w,), jnp.int32),
            pltpu.VMEM((b_per_w, EMBED_DIM), jnp.float32),
            pltpu.SemaphoreType.DMA,
        ],
    )
    def gather_kernel(table_hbm, idx_hbm, out_hbm, idx_v, rows_v, sem):
        wid = lax.axis_index("s") * info.num_cores + lax.axis_index("c")
        pltpu.sync_copy(idx_hbm.at[pl.ds(wid * b_per_w, b_per_w)], idx_v)
        pltpu.async_copy(table_hbm.at[idx_v], rows_v, sem).wait()
        pltpu.sync_copy(rows_v, out_hbm.at[pl.ds(wid * b_per_w, b_per_w)])

    return gather_kernel(codebook, idx)


def _dec2_body(zq0_ref, zq1_ref, wd1_ref, bd1_ref, wd2_ref, bd2_ref, xr_ref):
    i = pl.program_id(0)
    zq = jnp.where(i < NBH, zq0_ref[...], zq1_ref[...])
    h = _leaky(_dot(zq, wd1_ref[...]) + bd1_ref[...])
    xr_ref[...] = _dot(h, wd2_ref[...]) + bd2_ref[...]


def kernel(x, W_e1, b_e1, W_e2, b_e2, codebook, W_d1, b_d1, W_d2, b_d2):
    cn = jnp.sum(codebook * codebook, axis=1)[None, :]
    full = lambda shape: pl.BlockSpec(shape, lambda i: (0,) * len(shape))

    def enc_half(off):
        return pl.pallas_call(
            _enc_body,
            grid=(NBH,),
            in_specs=[
                pl.BlockSpec((BM, INPUT_DIM), lambda i: (i + off, 0)),
                full((INPUT_DIM, HIDDEN)),
                full((1, HIDDEN)),
                full((HIDDEN, EMBED_DIM)),
                full((1, EMBED_DIM)),
                full((NUM_EMB, EMBED_DIM)),
                full((1, NUM_EMB)),
            ],
            out_specs=[
                pl.BlockSpec((BM, 1), lambda i: (i, 0)),
                pl.BlockSpec((1, 1, 1), lambda i: (i, 0, 0)),
            ],
            out_shape=[
                jax.ShapeDtypeStruct((HALF, 1), jnp.int32),
                jax.ShapeDtypeStruct((NBH, 1, 1), jnp.float32),
            ],
            compiler_params=pltpu.CompilerParams(
                dimension_semantics=("parallel",)),
        )(x, W_e1, b_e1.reshape(1, -1), W_e2, b_e2.reshape(1, -1),
          codebook, cn)

    idx0, ls0 = enc_half(0)
    idx1, ls1 = enc_half(NBH)

    zq0 = _sc_gather(codebook, idx0.reshape(HALF))
    zq1 = _sc_gather(codebook, idx1.reshape(HALF))

    xr = pl.pallas_call(
        _dec2_body,
        grid=(NB,),
        in_specs=[
            pl.BlockSpec((BM, EMBED_DIM),
                         lambda i: (jnp.minimum(i, NBH - 1), 0)),
            pl.BlockSpec((BM, EMBED_DIM),
                         lambda i: (jnp.maximum(i, NBH) - NBH, 0)),
            full((EMBED_DIM, HIDDEN)),
            full((1, HIDDEN)),
            full((HIDDEN, INPUT_DIM)),
            full((1, INPUT_DIM)),
        ],
        out_specs=pl.BlockSpec((BM, INPUT_DIM), lambda i: (i, 0)),
        out_shape=jax.ShapeDtypeStruct((BATCH, INPUT_DIM), jnp.float32),
        compiler_params=pltpu.CompilerParams(
            dimension_semantics=("parallel",)),
    )(zq0, zq1, W_d1, b_d1.reshape(1, -1), W_d2, b_d2.reshape(1, -1))

    vq_loss = ((1.0 + COMMIT) * (jnp.sum(ls0) + jnp.sum(ls1))
               / (BATCH * EMBED_DIM))
    return xr, vq_loss


# speedup vs baseline: 1.0838x; 1.0838x over previous
import functools

import jax
import jax.numpy as jnp
from jax import lax
from jax.experimental import pallas as pl
from jax.experimental.pallas import tpu as pltpu
from jax.experimental.pallas import tpu_sc as plsc

INPUT_DIM = 768
EMBED_DIM = 256
HIDDEN = 512
NUM_EMB = 8192
BATCH = 4096
COMMIT = 0.25

BM = 512
NB = BATCH // BM
CHUNK = 1024
NCH = NUM_EMB // CHUNK

_BF = jnp.bfloat16


def _leaky(v):
    return jnp.where(v > 0, v, 0.01 * v)


def _dot(a, b):
    return jnp.dot(a.astype(_BF), b.astype(_BF),
                   preferred_element_type=jnp.float32)


def _enc_body(x_ref, we1_ref, be1_ref, we2_ref, be2_ref, cb_ref, cn_ref,
              idx_ref, ls_ref):
    h = _leaky(_dot(x_ref[...], we1_ref[...]) + be1_ref[...])
    z = _dot(h, we2_ref[...]) + be2_ref[...]
    znorm = jnp.sum(z * z, axis=1, keepdims=True)
    zm2b = (-2.0 * z).astype(_BF)
    runv = runc = None
    for c in range(NCH):
        sl = slice(c * CHUNK, (c + 1) * CHUNK)
        mm2 = lax.dot_general(zm2b, cb_ref[sl, :].astype(_BF),
                              (((1,), (1,)), ((), ())),
                              preferred_element_type=jnp.float32)
        s = (znorm + cn_ref[:, sl]) + mm2
        if c == 0:
            runv = s
            runc = jnp.zeros(s.shape, jnp.int32)
        else:
            upd = s < runv
            runv = jnp.where(upd, s, runv)
            runc = jnp.where(upd, jnp.int32(c), runc)
    gidx = runc * CHUNK + lax.broadcasted_iota(jnp.int32, runv.shape, 1)
    vmin = jnp.min(runv, axis=1, keepdims=True)
    cand = jnp.where(runv == vmin, gidx, jnp.int32(0x7FFFFFFF))
    idx_ref[...] = jnp.min(cand, axis=1)[:, None]
    ls_ref[...] = jnp.sum(vmin).reshape(1, 1, 1)


def _dec_body(zq_ref, wd1_ref, bd1_ref, wd2_ref, bd2_ref, xr_ref):
    h = _leaky(_dot(zq_ref[...], wd1_ref[...]) + bd1_ref[...])
    xr_ref[...] = _dot(h, wd2_ref[...]) + bd2_ref[...]


def _sc_gather(codebook, idx):
    info = plsc.get_sparse_core_info()
    nw = info.num_cores * info.num_subcores
    b_per_w = BATCH // nw
    mesh = plsc.VectorSubcoreMesh(core_axis_name="c", subcore_axis_name="s")

    @functools.partial(
        pl.kernel,
        out_type=jax.ShapeDtypeStruct((BATCH, EMBED_DIM), jnp.float32),
        mesh=mesh,
        scratch_types=[
            pltpu.VMEM((b_per_w,), jnp.int32),
            pltpu.VMEM((b_per_w, EMBED_DIM), jnp.float32),
            pltpu.SemaphoreType.DMA,
        ],
    )
    def gather_kernel(table_hbm, idx_hbm, out_hbm, idx_v, rows_v, sem):
        wid = lax.axis_index("s") * info.num_cores + lax.axis_index("c")
        pltpu.sync_copy(idx_hbm.at[pl.ds(wid * b_per_w, b_per_w)], idx_v)
        pltpu.async_copy(table_hbm.at[idx_v], rows_v, sem).wait()
        pltpu.sync_copy(rows_v, out_hbm.at[pl.ds(wid * b_per_w, b_per_w)])

    return gather_kernel(codebook, idx)


def kernel(x, W_e1, b_e1, W_e2, b_e2, codebook, W_d1, b_d1, W_d2, b_d2):
    cn = jnp.sum(codebook * codebook, axis=1)[None, :]
    full = lambda shape: pl.BlockSpec(shape, lambda i: (0,) * len(shape))

    idx, ls = pl.pallas_call(
        _enc_body,
        grid=(NB,),
        in_specs=[
            pl.BlockSpec((BM, INPUT_DIM), lambda i: (i, 0)),
            full((INPUT_DIM, HIDDEN)),
            full((1, HIDDEN)),
            full((HIDDEN, EMBED_DIM)),
            full((1, EMBED_DIM)),
            full((NUM_EMB, EMBED_DIM)),
            full((1, NUM_EMB)),
        ],
        out_specs=[
            pl.BlockSpec((BM, 1), lambda i: (i, 0)),
            pl.BlockSpec((1, 1, 1), lambda i: (i, 0, 0)),
        ],
        out_shape=[
            jax.ShapeDtypeStruct((BATCH, 1), jnp.int32),
            jax.ShapeDtypeStruct((NB, 1, 1), jnp.float32),
        ],
        compiler_params=pltpu.CompilerParams(
            dimension_semantics=("parallel",)),
    )(x, W_e1, b_e1.reshape(1, -1), W_e2, b_e2.reshape(1, -1), codebook, cn)

    zq = _sc_gather(codebook, idx.reshape(BATCH))

    xr = pl.pallas_call(
        _dec_body,
        grid=(NB,),
        in_specs=[
            pl.BlockSpec((BM, EMBED_DIM), lambda i: (i, 0)),
            full((EMBED_DIM, HIDDEN)),
            full((1, HIDDEN)),
            full((HIDDEN, INPUT_DIM)),
            full((1, INPUT_DIM)),
        ],
        out_specs=pl.BlockSpec((BM, INPUT_DIM), lambda i: (i, 0)),
        out_shape=jax.ShapeDtypeStruct((BATCH, INPUT_DIM), jnp.float32),
        compiler_params=pltpu.CompilerParams(
            dimension_semantics=("parallel",)),
    )(zq, W_d1, b_d1.reshape(1, -1), W_d2, b_d2.reshape(1, -1))

    vq_loss = (1.0 + COMMIT) * jnp.sum(ls) / (BATCH * EMBED_DIM)
    return xr, vq_loss


# speedup vs baseline: 1.1034x; 1.0181x over previous
import functools

import jax
import jax.numpy as jnp
from jax import lax
from jax.experimental import pallas as pl
from jax.experimental.pallas import tpu as pltpu
from jax.experimental.pallas import tpu_sc as plsc

INPUT_DIM = 768
EMBED_DIM = 256
HIDDEN = 512
NUM_EMB = 8192
BATCH = 4096
COMMIT = 0.25

BM = 512
NB = BATCH // BM
DBM = 1024
DNB = BATCH // DBM
CHUNK = 1024
NCH = NUM_EMB // CHUNK

_BF = jnp.bfloat16


def _leaky(v):
    return jnp.where(v > 0, v, 0.01 * v)


def _dot(a, b):
    return jnp.dot(a.astype(_BF), b.astype(_BF),
                   preferred_element_type=jnp.float32)


def _enc_body(x_ref, we1_ref, be1_ref, we2_ref, be2_ref, cb_ref, cn_ref,
              idx_ref, ls_ref):
    h = _leaky(_dot(x_ref[...], we1_ref[...]) + be1_ref[...])
    z = _dot(h, we2_ref[...]) + be2_ref[...]
    znorm = jnp.sum(z * z, axis=1, keepdims=True)
    zm2b = (-2.0 * z).astype(_BF)
    runv = runc = None
    for c in range(NCH):
        sl = slice(c * CHUNK, (c + 1) * CHUNK)
        mm2 = lax.dot_general(zm2b, cb_ref[sl, :].astype(_BF),
                              (((1,), (1,)), ((), ())),
                              preferred_element_type=jnp.float32)
        s = (znorm + cn_ref[:, sl]) + mm2
        if c == 0:
            runv = s
            runc = jnp.zeros(s.shape, jnp.int32)
        else:
            upd = s < runv
            runv = jnp.where(upd, s, runv)
            runc = jnp.where(upd, jnp.int32(c), runc)
    gidx = runc * CHUNK + lax.broadcasted_iota(jnp.int32, runv.shape, 1)
    vmin = jnp.min(runv, axis=1, keepdims=True)
    cand = jnp.where(runv == vmin, gidx, jnp.int32(0x7FFFFFFF))
    idx_ref[...] = jnp.min(cand, axis=1)[:, None]
    ls_ref[...] = jnp.sum(vmin).reshape(1, 1, 1)


def _dec_body(zq_ref, wd1_ref, bd1_ref, wd2_ref, bd2_ref, xr_ref):
    h = _leaky(_dot(zq_ref[...], wd1_ref[...]) + bd1_ref[...])
    xr_ref[...] = _dot(h, wd2_ref[...]) + bd2_ref[...]


def _sc_gather(codebook, idx):
    info = plsc.get_sparse_core_info()
    nw = info.num_cores * info.num_subcores
    b_per_w = BATCH // nw
    mesh = plsc.VectorSubcoreMesh(core_axis_name="c", subcore_axis_name="s")

    @functools.partial(
        pl.kernel,
        out_type=jax.ShapeDtypeStruct((BATCH, EMBED_DIM), jnp.float32),
        mesh=mesh,
        scratch_types=[
            pltpu.VMEM((b_per_w,), jnp.int32),
            pltpu.VMEM((b_per_w, EMBED_DIM), jnp.float32),
            pltpu.SemaphoreType.DMA,
        ],
    )
    def gather_kernel(table_hbm, idx_hbm, out_hbm, idx_v, rows_v, sem):
        wid = lax.axis_index("s") * info.num_cores + lax.axis_index("c")
        pltpu.sync_copy(idx_hbm.at[pl.ds(wid * b_per_w, b_per_w)], idx_v)
        pltpu.async_copy(table_hbm.at[idx_v], rows_v, sem).wait()
        pltpu.sync_copy(rows_v, out_hbm.at[pl.ds(wid * b_per_w, b_per_w)])

    return gather_kernel(codebook, idx)


def kernel(x, W_e1, b_e1, W_e2, b_e2, codebook, W_d1, b_d1, W_d2, b_d2):
    cn = jnp.sum(codebook * codebook, axis=1)[None, :]
    full = lambda shape: pl.BlockSpec(shape, lambda i: (0,) * len(shape))

    idx, ls = pl.pallas_call(
        _enc_body,
        grid=(NB,),
        in_specs=[
            pl.BlockSpec((BM, INPUT_DIM), lambda i: (i, 0)),
            full((INPUT_DIM, HIDDEN)),
            full((HIDDEN,)),
            full((HIDDEN, EMBED_DIM)),
            full((EMBED_DIM,)),
            full((NUM_EMB, EMBED_DIM)),
            full((1, NUM_EMB)),
        ],
        out_specs=[
            pl.BlockSpec((BM, 1), lambda i: (i, 0)),
            pl.BlockSpec((1, 1, 1), lambda i: (i, 0, 0)),
        ],
        out_shape=[
            jax.ShapeDtypeStruct((BATCH, 1), jnp.int32),
            jax.ShapeDtypeStruct((NB, 1, 1), jnp.float32),
        ],
        compiler_params=pltpu.CompilerParams(
            dimension_semantics=("parallel",)),
    )(x, W_e1, b_e1, W_e2, b_e2, codebook, cn)

    zq = _sc_gather(codebook, idx.reshape(BATCH))

    xr = pl.pallas_call(
        _dec_body,
        grid=(DNB,),
        in_specs=[
            pl.BlockSpec((DBM, EMBED_DIM), lambda i: (i, 0)),
            full((EMBED_DIM, HIDDEN)),
            full((HIDDEN,)),
            full((HIDDEN, INPUT_DIM)),
            full((INPUT_DIM,)),
        ],
        out_specs=pl.BlockSpec((DBM, INPUT_DIM), lambda i: (i, 0)),
        out_shape=jax.ShapeDtypeStruct((BATCH, INPUT_DIM), jnp.float32),
        compiler_params=pltpu.CompilerParams(
            dimension_semantics=("parallel",)),
    )(zq, W_d1, b_d1, W_d2, b_d2)

    vq_loss = (1.0 + COMMIT) * jnp.sum(ls) / (BATCH * EMBED_DIM)
    return xr, vq_loss
